# baseline (speedup 1.0000x reference)
import functools
import math

import jax
import jax.numpy as jnp
from jax import lax
from jax.experimental import pallas as pl
from jax.experimental.pallas import tpu as pltpu
from jax.experimental.pallas import tpu_sc as plsc

_NC = 2
_NS = 16
_NW = _NC * _NS
_L = 16


def _pad16(c):
    return ((c + 15) // 16) * 16


def _pad64(c):
    return ((c + 63) // 64) * 64


_MW = 120
_MB = 4 * (_MW + 12)


@functools.cache
def _make_sc_attn(n, cp, c_actual):
    tc = cp // 16
    npt = _pad16((n + _NW - 1) // _NW)
    inv = 1.0 / math.sqrt(float(c_actual))
    mesh = plsc.VectorSubcoreMesh(core_axis_name="c", subcore_axis_name="s")

    def body(q_hbm, kv_hbm, srcs_hbm, meta_hbm, tb_hbm, out_hbm,
             tbv, metabuf, srcwin, kvring, qring, aggbuf,
             swin_sem, kv_sem, q_sem):
        cid = lax.axis_index("c")
        sid = lax.axis_index("s")
        wid = sid * _NC + cid
        n_base = wid * npt
        pltpu.sync_copy(tb_hbm, tbv)
        bb = tbv[pl.ds(wid, 16)]
        b0, b1 = bb[0], bb[1]
        iota = lax.iota(jnp.int32, 16)
        nwin = lax.div(b1 - b0 + (_MW - 1), _MW)
        state0 = tuple(jnp.zeros((16,), jnp.float32) for _ in range(tc + 2))

        @pl.loop(0, nwin, init_carry=state0)
        def _win(w, state):
            gw0 = b0 + w * _MW
            cnt = jnp.minimum(_MW, b1 - gw0)
            fb_full = 4 * gw0
            fb = pl.multiple_of(fb_full & (-8), 8)
            off0 = fb_full - fb
            pltpu.sync_copy(meta_hbm.at[pl.ds(fb, _MB)], metabuf)

            def meta_at(idx):
                mv = metabuf[pl.ds(off0 + 4 * idx, 16)]
                return mv[0], mv[1], mv[2], mv[3]

            def issue_srcwin(l):
                _, e0l, _, _ = meta_at(l)
                basel = pl.multiple_of(e0l & (-8), 8)
                pltpu.make_async_copy(
                    srcs_hbm.at[pl.ds(basel, 32)],
                    srcwin.at[lax.bitwise_and(l, 3)], swin_sem).start()

            def wait_srcwin():
                pltpu.make_async_copy(
                    srcs_hbm.at[pl.ds(0, 32)], srcwin.at[0],
                    swin_sem).wait()

            def issue_kv_q(l):
                nodel, e0l, _, _ = meta_at(l)
                basel = pl.multiple_of(e0l & (-8), 8)
                slot = lax.bitwise_and(l, 3)
                sidx = srcwin[slot, pl.ds(e0l - basel, 16)]
                pltpu.make_async_copy(
                    kv_hbm.at[sidx], kvring.at[slot], kv_sem).start()
                pltpu.make_async_copy(
                    q_hbm.at[nodel], qring.at[slot], q_sem).start()

            for l in range(4):
                @pl.when(l < cnt)
                def _():
                    issue_srcwin(l)
            for l in range(2):
                @pl.when(l < cnt)
                def _():
                    wait_srcwin()
                    issue_kv_q(l)

            @pl.loop(0, cnt, init_carry=state)
            def _blk(i, carry):
                m, s = carry[0], carry[1]
                u = list(carry[2:])

                @pl.when(i + 4 < cnt)
                def _():
                    issue_srcwin(i + 4)

                @pl.when(i + 2 < cnt)
                def _():
                    wait_srcwin()
                    issue_kv_q(i + 2)

                pltpu.make_async_copy(
                    kv_hbm.at[pl.ds(0, 16)], kvring.at[0], kv_sem).wait()
                pltpu.make_async_copy(
                    q_hbm.at[0], qring.at[0], q_sem).wait()

                node, e0b, rem, fl = meta_at(i)
                slot = lax.bitwise_and(i, 3)
                first = lax.bitwise_and(fl, 1)
                last = fl // 2

                neg = jnp.full((16,), -3e38, jnp.float32)
                zero = jnp.zeros((16,), jnp.float32)
                m = jnp.where(first == 1, neg, m)
                s = jnp.where(first == 1, zero, s)
                u = [jnp.where(first == 1, zero, uu) for uu in u]

                qv = [qring[slot, pl.ds(t * 16, 16)] for t in range(tc)]
                mask = iota < rem
                alpha = jnp.full((16,), -1e30, jnp.float32)
                for r in range(16):
                    acc = qv[0] * kvring[slot, r, pl.ds(0, 16)]
                    for t in range(1, tc):
                        acc = acc + qv[t] * kvring[slot, r,
                                                   pl.ds(t * 16, 16)]
                    alpha = jnp.where(iota == r, jnp.sum(acc) * inv, alpha)
                alpha_m = jnp.where(mask, alpha, -1e30)
                m_new = jnp.maximum(m, jnp.max(alpha_m))
                scale = jnp.exp(m - m_new)
                evec = jnp.where(mask, jnp.exp(alpha - m_new), 0.0)
                s = s * scale + jnp.sum(evec)
                u = [uu * scale for uu in u]
                for r in range(16):
                    w_r = evec[r]
                    for t in range(tc):
                        u[t] = u[t] + w_r * kvring[slot, r,
                                                   pl.ds(cp + t * 16, 16)]

                @pl.when(last == 1)
                def _():
                    ln = node - n_base
                    row = lax.bitwise_and(ln, 15)
                    s_fin = s + 1e-16
                    for t in range(tc):
                        aggbuf[row, pl.ds(t * 16, 16)] = u[t] / s_fin

                    @pl.when(row == 15)
                    def _():
                        n0 = pl.multiple_of(n_base + ln - 15, 8)
                        pltpu.sync_copy(aggbuf, out_hbm.at[pl.ds(n0, 16)])

                return (m_new, s) + tuple(u)

            return _blk

    kfn = pl.kernel(
        body,
        out_type=jax.ShapeDtypeStruct((n, cp), jnp.float32),
        mesh=mesh,
        scratch_types=[
            pltpu.VMEM((48,), jnp.int32),
            pltpu.VMEM((_MB,), jnp.int32),
            pltpu.VMEM((4, 32), jnp.int32),
            pltpu.VMEM((4, 16, 2 * cp), jnp.float32),
            pltpu.VMEM((4, cp), jnp.float32),
            pltpu.VMEM((16, cp), jnp.float32),
            pltpu.SemaphoreType.DMA,
            pltpu.SemaphoreType.DMA,
            pltpu.SemaphoreType.DMA,
        ],
        compiler_params=pltpu.CompilerParams(needs_layout_passes=False),
    )
    return kfn, npt


_GRID = 10


def _stats_call(a, s, c, n, relu_first):
    blk = n // _GRID

    def body(a_ref, s_ref, o_ref):
        i = pl.program_id(0)
        z = a_ref[:, :c] + s_ref[:]
        if relu_first:
            z = jax.nn.relu(z)
        upd = jnp.concatenate(
            [jnp.sum(z, axis=0)[None], jnp.sum(z * z, axis=0)[None],
             jnp.zeros((6, c), jnp.float32)], axis=0)
        prev = jnp.where(i == 0, jnp.zeros((8, c), jnp.float32), o_ref[...])
        o_ref[...] = prev + upd

    return pl.pallas_call(
        body,
        grid=(_GRID,),
        in_specs=[
            pl.BlockSpec((blk, a.shape[1]), lambda i: (i, 0)),
            pl.BlockSpec((blk, c), lambda i: (i, 0)),
        ],
        out_specs=pl.BlockSpec((8, c), lambda i: (0, 0)),
        out_shape=jax.ShapeDtypeStruct((8, c), jnp.float32),
    )(a, s)


def _dense_call(prologue, ins, p, cp, n):
    fout = p["q"]["W"].shape[1]
    nw = len(ins)
    blk = n // _GRID

    def body(*refs):
        inr = refs[:nw]
        wq, bq, wk, bk, wv, bv, ws, bs = refs[nw:nw + 8]
        qo, kvo, so = refs[nw + 8:]
        h = prologue(*[r[...] for r in inr])
        q = jnp.dot(h, wq[:], preferred_element_type=jnp.float32) + bq[:]
        k = jnp.dot(h, wk[:], preferred_element_type=jnp.float32) + bk[:]
        v = jnp.dot(h, wv[:], preferred_element_type=jnp.float32) + bv[:]
        sk = jnp.dot(h, ws[:], preferred_element_type=jnp.float32) + bs[:]
        if cp > fout:
            z = jnp.zeros((blk, cp - fout), jnp.float32)
            qo[...] = jnp.concatenate([q, z], axis=1)
            kvo[...] = jnp.concatenate([k, z, v, z], axis=1)
        else:
            qo[...] = q
            kvo[...] = jnp.concatenate([k, v], axis=1)
        so[...] = sk

    def full_spec(arr):
        shp = arr.shape
        if len(shp) == 1:
            return pl.BlockSpec(shp, lambda i: (0,))
        return pl.BlockSpec(shp, lambda i: (0,) * len(shp))

    in_specs = []
    args = []
    for arr, kind in ins:
        args.append(arr)
        if kind == "rows":
            in_specs.append(
                pl.BlockSpec((blk, arr.shape[1]), lambda i: (i, 0)))
        else:
            in_specs.append(full_spec(arr))
    for w in [p["q"]["W"], p["q"]["b"], p["k"]["W"], p["k"]["b"],
              p["v"]["W"], p["v"]["b"], p["skip"]["W"], p["skip"]["b"]]:
        args.append(w)
        in_specs.append(full_spec(w))

    out_shape = [
        jax.ShapeDtypeStruct((n, cp), jnp.float32),
        jax.ShapeDtypeStruct((n, 2 * cp), jnp.float32),
        jax.ShapeDtypeStruct((n, fout), jnp.float32),
    ]
    out_specs = [
        pl.BlockSpec((blk, cp), lambda i: (i, 0)),
        pl.BlockSpec((blk, 2 * cp), lambda i: (i, 0)),
        pl.BlockSpec((blk, fout), lambda i: (i, 0)),
    ]
    return pl.pallas_call(
        body, grid=(_GRID,), in_specs=in_specs, out_specs=out_specs,
        out_shape=out_shape)(*args)


def _bn_apply(z, g, b, stats, n, eps=1e-5):
    mu = stats[0] / n
    var = stats[1] / n - mu * mu
    return (z - mu) / jnp.sqrt(var + eps) * g + b


def _logsoftmax_call(agg, sk, c, n):
    def body(a_ref, s_ref, o_ref):
        z = a_ref[:, :c] + s_ref[:]
        m = jnp.max(z, axis=1, keepdims=True)
        lse = m + jnp.log(jnp.sum(jnp.exp(z - m), axis=1, keepdims=True))
        o_ref[:] = z - lse

    return pl.pallas_call(
        body, out_shape=jax.ShapeDtypeStruct((n, c), jnp.float32))(agg, sk)


def kernel(x, edge_index, params):
    n = x.shape[0]
    e = edge_index.shape[1]

    shift = max(n - 1, e // n).bit_length()
    packed = lax.sort(
        (edge_index[1] << shift) | edge_index[0], dimension=0)
    dsts = packed >> shift
    srcs = packed & ((1 << shift) - 1)
    offs = jnp.searchsorted(dsts, jnp.arange(n + 1, dtype=jnp.int32),
                            method="scan_unrolled").astype(jnp.int32)
    srcs_p = jnp.concatenate([srcs, jnp.zeros((64,), jnp.int32)])

    npt = _pad16((n + _NW - 1) // _NW)
    gmax = e // 16 + n
    deg = offs[1:] - offs[:-1]
    nblk = jnp.maximum((deg + 15) // 16, 1)
    boff = jnp.concatenate(
        [jnp.zeros((1,), jnp.int32),
         jnp.cumsum(nblk).astype(jnp.int32)])
    g = jnp.arange(gmax, dtype=jnp.int32)
    node = jnp.minimum(
        jnp.searchsorted(boff, g, side="right",
                         method="scan_unrolled").astype(jnp.int32) - 1,
        n - 1)
    st = g - boff[node]
    valid = g < boff[n]
    e0 = jnp.where(valid, offs[node] + st * 16, 0)
    rem = jnp.where(valid, jnp.clip(deg[node] - st * 16, 0, 16), 0)
    fl = jnp.where(valid,
                   (st == 0).astype(jnp.int32)
                   + 2 * (st == nblk[node] - 1).astype(jnp.int32), 0)
    meta = jnp.stack([node, e0, rem, fl], axis=1).reshape(-1)
    meta = jnp.concatenate([meta, jnp.zeros((4 * (_MW + 16),), jnp.int32)])
    tb = boff[jnp.minimum(
        jnp.arange(33, dtype=jnp.int32) * npt, n)].astype(jnp.int32)
    tb = jnp.concatenate([tb, jnp.full((15,), boff[n], jnp.int32)])

    def attn(hq, hkv, p, c_actual):
        cp = hq.shape[1]
        kfn, _ = _make_sc_attn(n, cp, c_actual)
        return kfn(hq, hkv, srcs_p, meta, tb)

    p = params

    def bn_ins(agg, sk, bn, c, relu_first=False):
        st = _stats_call(agg, sk, c, n, relu_first)
        return [(agg, "rows"), (sk, "rows"), (bn["g"], "full"),
                (bn["b"], "full"), (st, "full")]

    q1, kv1, s1 = _dense_call(lambda a: a, [(x, "rows")], p["conv1"],
                              _pad64(220), n)
    agg1 = attn(q1, kv1, p["conv1"], 220)

    q2, kv2, s2 = _dense_call(
        lambda a, s, g, b, st: jax.nn.relu(
            _bn_apply(a[:, :220] + s, g, b, st, n)),
        bn_ins(agg1, s1, p["bn1"], 220), p["conv2"], _pad64(150), n)
    agg2 = attn(q2, kv2, p["conv2"], 150)

    qs, kvs, ss = _dense_call(lambda a: a, [(x, "rows")], p["skip1"],
                              _pad64(150), n)
    aggs = attn(qs, kvs, p["skip1"], 150)

    q3, kv3, s3 = _dense_call(
        lambda a, s, g, b, st, ai, si: jax.nn.relu(
            _bn_apply(a[:, :150] + s, g, b, st, n) + ai[:, :150] + si),
        bn_ins(agg2, s2, p["bn2"], 150) + [(aggs, "rows"), (ss, "rows")],
        p["conv3"], _pad64(100), n)
    agg3 = attn(q3, kv3, p["conv3"], 100)

    q4, kv4, s4 = _dense_call(
        lambda a, s, g, b, st: jax.nn.relu(
            _bn_apply(a[:, :100] + s, g, b, st, n)),
        bn_ins(agg3, s3, p["bn3"], 100), p["conv4"], _pad64(60), n)
    agg4 = attn(q4, kv4, p["conv4"], 60)

    q5, kv5, s5 = _dense_call(
        lambda a, s, g, b, st: jax.nn.relu(
            _bn_apply(a[:, :60] + s, g, b, st, n)),
        bn_ins(agg4, s4, p["bn4"], 60), p["conv5"], _pad64(30), n)
    agg5 = attn(q5, kv5, p["conv5"], 30)

    q6, kv6, s6 = _dense_call(
        lambda a, s, g, b, st: _bn_apply(
            jax.nn.relu(a[:, :30] + s), g, b, st, n),
        bn_ins(agg5, s5, p["bn5"], 30, relu_first=True), p["conv6"],
        _pad64(10), n)
    agg6 = attn(q6, kv6, p["conv6"], 10)

    return _logsoftmax_call(agg6, s6, 10, n)


# speedup vs baseline: 1.3675x; 1.3675x over previous
import functools
import math

import jax
import jax.numpy as jnp
from jax import lax
from jax.experimental import pallas as pl
from jax.experimental.pallas import tpu as pltpu
from jax.experimental.pallas import tpu_sc as plsc

_NC = 2
_NS = 16
_NW = _NC * _NS
_L = 16


def _pad16(c):
    return ((c + 15) // 16) * 16


def _pad64(c):
    return ((c + 63) // 64) * 64


_MW = 120
_MB = 4 * (_MW + 12)


@functools.cache
def _make_sc_attn(n, cp, c_actual):
    tc = cp // 16
    npt = _pad16((n + _NW - 1) // _NW)
    inv = 1.0 / math.sqrt(float(c_actual))
    mesh = plsc.VectorSubcoreMesh(core_axis_name="c", subcore_axis_name="s")

    def body(q_hbm, kv_hbm, srcs_hbm, meta_hbm, tb_hbm, out_hbm,
             tbv, metabuf, srcwin, kvring, qring, aggbuf,
             swin_sem, kv_sem, q_sem):
        cid = lax.axis_index("c")
        sid = lax.axis_index("s")
        wid = sid * _NC + cid
        n_base = wid * npt
        pltpu.sync_copy(tb_hbm, tbv)
        bb = tbv[pl.ds(wid, 16)]
        b0, b1 = bb[0], bb[1]
        iota = lax.iota(jnp.int32, 16)
        nwin = lax.div(b1 - b0 + (_MW - 1), _MW)
        state0 = tuple(jnp.zeros((16,), jnp.float32) for _ in range(tc + 2))

        @pl.loop(0, nwin, init_carry=state0)
        def _win(w, state):
            gw0 = b0 + w * _MW
            cnt = jnp.minimum(_MW, b1 - gw0)
            fb_full = 4 * gw0
            fb = pl.multiple_of(fb_full & (-8), 8)
            off0 = fb_full - fb
            pltpu.sync_copy(meta_hbm.at[pl.ds(fb, _MB)], metabuf)

            def meta_at(idx):
                mv = metabuf[pl.ds(off0 + 4 * idx, 16)]
                return mv[0], mv[1], mv[2], mv[3]

            def issue_srcwin(l):
                _, e0l, _, _ = meta_at(l)
                basel = pl.multiple_of(e0l & (-8), 8)
                pltpu.make_async_copy(
                    srcs_hbm.at[pl.ds(basel, 32)],
                    srcwin.at[lax.bitwise_and(l, 3)], swin_sem).start()

            def wait_srcwin():
                pltpu.make_async_copy(
                    srcs_hbm.at[pl.ds(0, 32)], srcwin.at[0],
                    swin_sem).wait()

            def issue_kv_q(l):
                nodel, e0l, _, _ = meta_at(l)
                basel = pl.multiple_of(e0l & (-8), 8)
                slot = lax.bitwise_and(l, 3)
                sidx = srcwin[slot, pl.ds(e0l - basel, 16)]
                pltpu.make_async_copy(
                    kv_hbm.at[sidx], kvring.at[slot], kv_sem).start()
                pltpu.make_async_copy(
                    q_hbm.at[nodel], qring.at[slot], q_sem).start()

            for l in range(4):
                @pl.when(l < cnt)
                def _():
                    issue_srcwin(l)
            for l in range(2):
                @pl.when(l < cnt)
                def _():
                    wait_srcwin()
                    issue_kv_q(l)

            @pl.loop(0, cnt, init_carry=state)
            def _blk(i, carry):
                m, s = carry[0], carry[1]
                u = list(carry[2:])

                @pl.when(i + 4 < cnt)
                def _():
                    issue_srcwin(i + 4)

                @pl.when(i + 2 < cnt)
                def _():
                    wait_srcwin()
                    issue_kv_q(i + 2)

                pltpu.make_async_copy(
                    kv_hbm.at[pl.ds(0, 16)], kvring.at[0], kv_sem).wait()
                pltpu.make_async_copy(
                    q_hbm.at[0], qring.at[0], q_sem).wait()

                node, e0b, rem, fl = meta_at(i)
                slot = lax.bitwise_and(i, 3)
                first = lax.bitwise_and(fl, 1)
                last = fl // 2

                neg = jnp.full((16,), -3e38, jnp.float32)
                zero = jnp.zeros((16,), jnp.float32)
                m = jnp.where(first == 1, neg, m)
                s = jnp.where(first == 1, zero, s)
                u = [jnp.where(first == 1, zero, uu) for uu in u]

                qv = [qring[slot, pl.ds(t * 16, 16)] for t in range(tc)]
                mask = iota < rem
                alpha = jnp.full((16,), -1e30, jnp.float32)
                for r in range(16):
                    acc = qv[0] * kvring[slot, r, pl.ds(0, 16)]
                    for t in range(1, tc):
                        acc = acc + qv[t] * kvring[slot, r,
                                                   pl.ds(t * 16, 16)]
                    alpha = jnp.where(iota == r, jnp.sum(acc) * inv, alpha)
                alpha_m = jnp.where(mask, alpha, -1e30)
                m_new = jnp.maximum(m, jnp.max(alpha_m))
                scale = jnp.exp(m - m_new)
                evec = jnp.where(mask, jnp.exp(alpha - m_new), 0.0)
                s = s * scale + jnp.sum(evec)
                u = [uu * scale for uu in u]
                for r in range(16):
                    w_r = evec[r]
                    for t in range(tc):
                        u[t] = u[t] + w_r * kvring[slot, r,
                                                   pl.ds(cp + t * 16, 16)]

                @pl.when(last == 1)
                def _():
                    ln = node - n_base
                    row = lax.bitwise_and(ln, 15)
                    s_fin = s + 1e-16
                    for t in range(tc):
                        aggbuf[row, pl.ds(t * 16, 16)] = u[t] / s_fin

                    @pl.when(row == 15)
                    def _():
                        n0 = pl.multiple_of(n_base + ln - 15, 8)
                        pltpu.sync_copy(aggbuf, out_hbm.at[pl.ds(n0, 16)])

                return (m_new, s) + tuple(u)

            return _blk

    kfn = pl.kernel(
        body,
        out_type=jax.ShapeDtypeStruct((n, cp), jnp.float32),
        mesh=mesh,
        scratch_types=[
            pltpu.VMEM((48,), jnp.int32),
            pltpu.VMEM((_MB,), jnp.int32),
            pltpu.VMEM((4, 32), jnp.int32),
            pltpu.VMEM((4, 16, 2 * cp), jnp.float32),
            pltpu.VMEM((4, cp), jnp.float32),
            pltpu.VMEM((16, cp), jnp.float32),
            pltpu.SemaphoreType.DMA,
            pltpu.SemaphoreType.DMA,
            pltpu.SemaphoreType.DMA,
        ],
        compiler_params=pltpu.CompilerParams(needs_layout_passes=False),
    )
    return kfn, npt


_GRID = 10


def _stats_call(a, s, c, n, relu_first):
    blk = n // _GRID

    def body(a_ref, s_ref, o_ref):
        i = pl.program_id(0)
        z = a_ref[:, :c] + s_ref[:]
        if relu_first:
            z = jax.nn.relu(z)
        upd = jnp.concatenate(
            [jnp.sum(z, axis=0)[None], jnp.sum(z * z, axis=0)[None],
             jnp.zeros((6, c), jnp.float32)], axis=0)
        prev = jnp.where(i == 0, jnp.zeros((8, c), jnp.float32), o_ref[...])
        o_ref[...] = prev + upd

    return pl.pallas_call(
        body,
        grid=(_GRID,),
        in_specs=[
            pl.BlockSpec((blk, a.shape[1]), lambda i: (i, 0)),
            pl.BlockSpec((blk, c), lambda i: (i, 0)),
        ],
        out_specs=pl.BlockSpec((8, c), lambda i: (0, 0)),
        out_shape=jax.ShapeDtypeStruct((8, c), jnp.float32),
    )(a, s)


def _dense_call(prologue, ins, p, cp, n):
    fout = p["q"]["W"].shape[1]
    nw = len(ins)
    blk = n // _GRID

    def body(*refs):
        inr = refs[:nw]
        wq, bq, wk, bk, wv, bv, ws, bs = refs[nw:nw + 8]
        qo, kvo, so = refs[nw + 8:]
        h = prologue(*[r[...] for r in inr])
        q = jnp.dot(h, wq[:], preferred_element_type=jnp.float32) + bq[:]
        k = jnp.dot(h, wk[:], preferred_element_type=jnp.float32) + bk[:]
        v = jnp.dot(h, wv[:], preferred_element_type=jnp.float32) + bv[:]
        sk = jnp.dot(h, ws[:], preferred_element_type=jnp.float32) + bs[:]
        if cp > fout:
            z = jnp.zeros((blk, cp - fout), jnp.float32)
            qo[...] = jnp.concatenate([q, z], axis=1)
            kvo[...] = jnp.concatenate([k, z, v, z], axis=1)
        else:
            qo[...] = q
            kvo[...] = jnp.concatenate([k, v], axis=1)
        so[...] = sk

    def full_spec(arr):
        shp = arr.shape
        if len(shp) == 1:
            return pl.BlockSpec(shp, lambda i: (0,))
        return pl.BlockSpec(shp, lambda i: (0,) * len(shp))

    in_specs = []
    args = []
    for arr, kind in ins:
        args.append(arr)
        if kind == "rows":
            in_specs.append(
                pl.BlockSpec((blk, arr.shape[1]), lambda i: (i, 0)))
        else:
            in_specs.append(full_spec(arr))
    for w in [p["q"]["W"], p["q"]["b"], p["k"]["W"], p["k"]["b"],
              p["v"]["W"], p["v"]["b"], p["skip"]["W"], p["skip"]["b"]]:
        args.append(w)
        in_specs.append(full_spec(w))

    out_shape = [
        jax.ShapeDtypeStruct((n, cp), jnp.float32),
        jax.ShapeDtypeStruct((n, 2 * cp), jnp.float32),
        jax.ShapeDtypeStruct((n, fout), jnp.float32),
    ]
    out_specs = [
        pl.BlockSpec((blk, cp), lambda i: (i, 0)),
        pl.BlockSpec((blk, 2 * cp), lambda i: (i, 0)),
        pl.BlockSpec((blk, fout), lambda i: (i, 0)),
    ]
    return pl.pallas_call(
        body, grid=(_GRID,), in_specs=in_specs, out_specs=out_specs,
        out_shape=out_shape)(*args)


def _bn_apply(z, g, b, stats, n, eps=1e-5):
    mu = stats[0] / n
    var = stats[1] / n - mu * mu
    return (z - mu) / jnp.sqrt(var + eps) * g + b


def _logsoftmax_call(agg, sk, c, n):
    def body(a_ref, s_ref, o_ref):
        z = a_ref[:, :c] + s_ref[:]
        m = jnp.max(z, axis=1, keepdims=True)
        lse = m + jnp.log(jnp.sum(jnp.exp(z - m), axis=1, keepdims=True))
        o_ref[:] = z - lse

    return pl.pallas_call(
        body, out_shape=jax.ShapeDtypeStruct((n, c), jnp.float32))(agg, sk)


def kernel(x, edge_index, params):
    n = x.shape[0]
    e = edge_index.shape[1]

    shift = max(n - 1, e // n).bit_length()
    packed = lax.sort(
        (edge_index[1] << shift) | edge_index[0], dimension=0)
    dsts = packed >> shift
    srcs = packed & ((1 << shift) - 1)
    offs = jnp.searchsorted(dsts, jnp.arange(n + 1, dtype=jnp.int32),
                            method="scan_unrolled").astype(jnp.int32)
    srcs_p = jnp.concatenate([srcs, jnp.zeros((64,), jnp.int32)])

    npt = _pad16((n + _NW - 1) // _NW)
    gmax = e // 16 + n
    deg = offs[1:] - offs[:-1]
    nblk = jnp.maximum((deg + 15) // 16, 1)
    boff = jnp.concatenate(
        [jnp.zeros((1,), jnp.int32),
         jnp.cumsum(nblk).astype(jnp.int32)])
    g = jnp.arange(gmax, dtype=jnp.int32)
    node = jnp.minimum(
        jnp.searchsorted(boff, g, side="right",
                         method="sort").astype(jnp.int32) - 1,
        n - 1)
    st = g - boff[node]
    valid = g < boff[n]
    e0 = jnp.where(valid, offs[node] + st * 16, 0)
    rem = jnp.where(valid, jnp.clip(deg[node] - st * 16, 0, 16), 0)
    fl = jnp.where(valid,
                   (st == 0).astype(jnp.int32)
                   + 2 * (st == nblk[node] - 1).astype(jnp.int32), 0)
    meta = jnp.stack([node, e0, rem, fl], axis=1).reshape(-1)
    meta = jnp.concatenate([meta, jnp.zeros((4 * (_MW + 16),), jnp.int32)])
    tb = boff[jnp.minimum(
        jnp.arange(33, dtype=jnp.int32) * npt, n)].astype(jnp.int32)
    tb = jnp.concatenate([tb, jnp.full((15,), boff[n], jnp.int32)])

    def attn(hq, hkv, p, c_actual):
        cp = hq.shape[1]
        kfn, _ = _make_sc_attn(n, cp, c_actual)
        return kfn(hq, hkv, srcs_p, meta, tb)

    p = params

    def bn_ins(agg, sk, bn, c, relu_first=False):
        st = _stats_call(agg, sk, c, n, relu_first)
        return [(agg, "rows"), (sk, "rows"), (bn["g"], "full"),
                (bn["b"], "full"), (st, "full")]

    q1, kv1, s1 = _dense_call(lambda a: a, [(x, "rows")], p["conv1"],
                              _pad64(220), n)
    agg1 = attn(q1, kv1, p["conv1"], 220)

    q2, kv2, s2 = _dense_call(
        lambda a, s, g, b, st: jax.nn.relu(
            _bn_apply(a[:, :220] + s, g, b, st, n)),
        bn_ins(agg1, s1, p["bn1"], 220), p["conv2"], _pad64(150), n)
    agg2 = attn(q2, kv2, p["conv2"], 150)

    qs, kvs, ss = _dense_call(lambda a: a, [(x, "rows")], p["skip1"],
                              _pad64(150), n)
    aggs = attn(qs, kvs, p["skip1"], 150)

    q3, kv3, s3 = _dense_call(
        lambda a, s, g, b, st, ai, si: jax.nn.relu(
            _bn_apply(a[:, :150] + s, g, b, st, n) + ai[:, :150] + si),
        bn_ins(agg2, s2, p["bn2"], 150) + [(aggs, "rows"), (ss, "rows")],
        p["conv3"], _pad64(100), n)
    agg3 = attn(q3, kv3, p["conv3"], 100)

    q4, kv4, s4 = _dense_call(
        lambda a, s, g, b, st: jax.nn.relu(
            _bn_apply(a[:, :100] + s, g, b, st, n)),
        bn_ins(agg3, s3, p["bn3"], 100), p["conv4"], _pad64(60), n)
    agg4 = attn(q4, kv4, p["conv4"], 60)

    q5, kv5, s5 = _dense_call(
        lambda a, s, g, b, st: jax.nn.relu(
            _bn_apply(a[:, :60] + s, g, b, st, n)),
        bn_ins(agg4, s4, p["bn4"], 60), p["conv5"], _pad64(30), n)
    agg5 = attn(q5, kv5, p["conv5"], 30)

    q6, kv6, s6 = _dense_call(
        lambda a, s, g, b, st: _bn_apply(
            jax.nn.relu(a[:, :30] + s), g, b, st, n),
        bn_ins(agg5, s5, p["bn5"], 30, relu_first=True), p["conv6"],
        _pad64(10), n)
    agg6 = attn(q6, kv6, p["conv6"], 10)

    return _logsoftmax_call(agg6, s6, 10, n)


# speedup vs baseline: 1.4275x; 1.0439x over previous
import functools
import math

import jax
import jax.numpy as jnp
from jax import lax
from jax.experimental import pallas as pl
from jax.experimental.pallas import tpu as pltpu
from jax.experimental.pallas import tpu_sc as plsc

_NC = 2
_NS = 16
_NW = _NC * _NS
_L = 16


def _pad16(c):
    return ((c + 15) // 16) * 16


def _pad64(c):
    return ((c + 63) // 64) * 64


_MW = 120
_MB = 4 * (_MW + 12)


@functools.cache
def _make_sc_attn(n, cp, c_actual):
    tc = cp // 16
    npt = _pad16((n + _NW - 1) // _NW)
    inv = 1.0 / math.sqrt(float(c_actual))
    mesh = plsc.VectorSubcoreMesh(core_axis_name="c", subcore_axis_name="s")

    def body(q_hbm, kv_hbm, srcs_hbm, meta_hbm, tb_hbm, out_hbm,
             tbv, metabuf, srcwin, kvring, qring, aggbuf,
             swin_sem, kv_sem, q_sem):
        cid = lax.axis_index("c")
        sid = lax.axis_index("s")
        wid = sid * _NC + cid
        n_base = wid * npt
        pltpu.sync_copy(tb_hbm, tbv)
        bb = tbv[pl.ds(wid, 16)]
        b0, b1 = bb[0], bb[1]
        iota = lax.iota(jnp.int32, 16)
        nwin = lax.div(b1 - b0 + (_MW - 1), _MW)
        state0 = tuple(jnp.zeros((16,), jnp.float32) for _ in range(tc + 2))

        @pl.loop(0, nwin, init_carry=state0)
        def _win(w, state):
            gw0 = b0 + w * _MW
            cnt = jnp.minimum(_MW, b1 - gw0)
            fb_full = 4 * gw0
            fb = pl.multiple_of(fb_full & (-8), 8)
            off0 = fb_full - fb
            pltpu.sync_copy(meta_hbm.at[pl.ds(fb, _MB)], metabuf)

            def meta_at(idx):
                mv = metabuf[pl.ds(off0 + 4 * idx, 16)]
                return mv[0], mv[1], mv[2], mv[3]

            def issue_srcwin(l):
                _, e0l, _, _ = meta_at(l)
                basel = pl.multiple_of(e0l & (-8), 8)
                pltpu.make_async_copy(
                    srcs_hbm.at[pl.ds(basel, 32)],
                    srcwin.at[lax.bitwise_and(l, 3)], swin_sem).start()

            def wait_srcwin():
                pltpu.make_async_copy(
                    srcs_hbm.at[pl.ds(0, 32)], srcwin.at[0],
                    swin_sem).wait()

            def issue_kv_q(l):
                nodel, e0l, _, _ = meta_at(l)
                basel = pl.multiple_of(e0l & (-8), 8)
                slot = lax.bitwise_and(l, 3)
                sidx = srcwin[slot, pl.ds(e0l - basel, 16)]
                pltpu.make_async_copy(
                    kv_hbm.at[sidx], kvring.at[slot], kv_sem).start()
                pltpu.make_async_copy(
                    q_hbm.at[nodel], qring.at[slot], q_sem).start()

            for l in range(4):
                @pl.when(l < cnt)
                def _():
                    issue_srcwin(l)
            for l in range(2):
                @pl.when(l < cnt)
                def _():
                    wait_srcwin()
                    issue_kv_q(l)

            @pl.loop(0, cnt, init_carry=state)
            def _blk(i, carry):
                m, s = carry[0], carry[1]
                u = list(carry[2:])

                @pl.when(i + 4 < cnt)
                def _():
                    issue_srcwin(i + 4)

                @pl.when(i + 2 < cnt)
                def _():
                    wait_srcwin()
                    issue_kv_q(i + 2)

                pltpu.make_async_copy(
                    kv_hbm.at[pl.ds(0, 16)], kvring.at[0], kv_sem).wait()
                pltpu.make_async_copy(
                    q_hbm.at[0], qring.at[0], q_sem).wait()

                node, e0b, rem, fl = meta_at(i)
                slot = lax.bitwise_and(i, 3)
                first = lax.bitwise_and(fl, 1)
                last = fl // 2

                neg = jnp.full((16,), -3e38, jnp.float32)
                zero = jnp.zeros((16,), jnp.float32)
                m = jnp.where(first == 1, neg, m)
                s = jnp.where(first == 1, zero, s)
                u = [jnp.where(first == 1, zero, uu) for uu in u]

                qv = [qring[slot, pl.ds(t * 16, 16)] for t in range(tc)]
                mask = iota < rem
                alpha = jnp.full((16,), -1e30, jnp.float32)
                for r in range(16):
                    acc = qv[0] * kvring[slot, r, pl.ds(0, 16)]
                    for t in range(1, tc):
                        acc = acc + qv[t] * kvring[slot, r,
                                                   pl.ds(t * 16, 16)]
                    alpha = jnp.where(iota == r, jnp.sum(acc) * inv, alpha)
                alpha_m = jnp.where(mask, alpha, -1e30)
                m_new = jnp.maximum(m, jnp.max(alpha_m))
                scale = jnp.exp(m - m_new)
                evec = jnp.where(mask, jnp.exp(alpha - m_new), 0.0)
                s = s * scale + jnp.sum(evec)
                u = [uu * scale for uu in u]
                for r in range(16):
                    w_r = evec[r]
                    for t in range(tc):
                        u[t] = u[t] + w_r * kvring[slot, r,
                                                   pl.ds(cp + t * 16, 16)]

                @pl.when(last == 1)
                def _():
                    ln = node - n_base
                    row = lax.bitwise_and(ln, 15)
                    s_fin = s + 1e-16
                    for t in range(tc):
                        aggbuf[row, pl.ds(t * 16, 16)] = u[t] / s_fin

                    @pl.when(row == 15)
                    def _():
                        n0 = pl.multiple_of(n_base + ln - 15, 8)
                        pltpu.sync_copy(aggbuf, out_hbm.at[pl.ds(n0, 16)])

                return (m_new, s) + tuple(u)

            return _blk

    kfn = pl.kernel(
        body,
        out_type=jax.ShapeDtypeStruct((n, cp), jnp.float32),
        mesh=mesh,
        scratch_types=[
            pltpu.VMEM((48,), jnp.int32),
            pltpu.VMEM((_MB,), jnp.int32),
            pltpu.VMEM((4, 32), jnp.int32),
            pltpu.VMEM((4, 16, 2 * cp), jnp.float32),
            pltpu.VMEM((4, cp), jnp.float32),
            pltpu.VMEM((16, cp), jnp.float32),
            pltpu.SemaphoreType.DMA,
            pltpu.SemaphoreType.DMA,
            pltpu.SemaphoreType.DMA,
        ],
        compiler_params=pltpu.CompilerParams(needs_layout_passes=False),
    )
    return kfn, npt


_GRID = 10


def _stats_call(a, s, c, n, relu_first):
    blk = n // _GRID

    def body(a_ref, s_ref, o_ref):
        i = pl.program_id(0)
        z = a_ref[:, :c] + s_ref[:]
        if relu_first:
            z = jax.nn.relu(z)
        upd = jnp.concatenate(
            [jnp.sum(z, axis=0)[None], jnp.sum(z * z, axis=0)[None],
             jnp.zeros((6, c), jnp.float32)], axis=0)
        prev = jnp.where(i == 0, jnp.zeros((8, c), jnp.float32), o_ref[...])
        o_ref[...] = prev + upd

    return pl.pallas_call(
        body,
        grid=(_GRID,),
        in_specs=[
            pl.BlockSpec((blk, a.shape[1]), lambda i: (i, 0)),
            pl.BlockSpec((blk, c), lambda i: (i, 0)),
        ],
        out_specs=pl.BlockSpec((8, c), lambda i: (0, 0)),
        out_shape=jax.ShapeDtypeStruct((8, c), jnp.float32),
    )(a, s)


def _dense_call(prologue, ins, p, cp, n):
    fout = p["q"]["W"].shape[1]
    nw = len(ins)
    blk = n // _GRID

    def body(*refs):
        inr = refs[:nw]
        wq, bq, wk, bk, wv, bv, ws, bs = refs[nw:nw + 8]
        qo, kvo, so = refs[nw + 8:]
        h = prologue(*[r[...] for r in inr])
        q = jnp.dot(h, wq[:], preferred_element_type=jnp.float32) + bq[:]
        k = jnp.dot(h, wk[:], preferred_element_type=jnp.float32) + bk[:]
        v = jnp.dot(h, wv[:], preferred_element_type=jnp.float32) + bv[:]
        sk = jnp.dot(h, ws[:], preferred_element_type=jnp.float32) + bs[:]
        if cp > fout:
            z = jnp.zeros((blk, cp - fout), jnp.float32)
            qo[...] = jnp.concatenate([q, z], axis=1)
            kvo[...] = jnp.concatenate([k, z, v, z], axis=1)
        else:
            qo[...] = q
            kvo[...] = jnp.concatenate([k, v], axis=1)
        so[...] = sk

    def full_spec(arr):
        shp = arr.shape
        if len(shp) == 1:
            return pl.BlockSpec(shp, lambda i: (0,))
        return pl.BlockSpec(shp, lambda i: (0,) * len(shp))

    in_specs = []
    args = []
    for arr, kind in ins:
        args.append(arr)
        if kind == "rows":
            in_specs.append(
                pl.BlockSpec((blk, arr.shape[1]), lambda i: (i, 0)))
        else:
            in_specs.append(full_spec(arr))
    for w in [p["q"]["W"], p["q"]["b"], p["k"]["W"], p["k"]["b"],
              p["v"]["W"], p["v"]["b"], p["skip"]["W"], p["skip"]["b"]]:
        args.append(w)
        in_specs.append(full_spec(w))

    out_shape = [
        jax.ShapeDtypeStruct((n, cp), jnp.float32),
        jax.ShapeDtypeStruct((n, 2 * cp), jnp.float32),
        jax.ShapeDtypeStruct((n, fout), jnp.float32),
    ]
    out_specs = [
        pl.BlockSpec((blk, cp), lambda i: (i, 0)),
        pl.BlockSpec((blk, 2 * cp), lambda i: (i, 0)),
        pl.BlockSpec((blk, fout), lambda i: (i, 0)),
    ]
    return pl.pallas_call(
        body, grid=(_GRID,), in_specs=in_specs, out_specs=out_specs,
        out_shape=out_shape)(*args)


def _bn_apply(z, g, b, stats, n, eps=1e-5):
    mu = stats[0] / n
    var = stats[1] / n - mu * mu
    return (z - mu) / jnp.sqrt(var + eps) * g + b


def _logsoftmax_call(agg, sk, c, n):
    def body(a_ref, s_ref, o_ref):
        z = a_ref[:, :c] + s_ref[:]
        m = jnp.max(z, axis=1, keepdims=True)
        lse = m + jnp.log(jnp.sum(jnp.exp(z - m), axis=1, keepdims=True))
        o_ref[:] = z - lse

    return pl.pallas_call(
        body, out_shape=jax.ShapeDtypeStruct((n, c), jnp.float32))(agg, sk)


def kernel(x, edge_index, params):
    n = x.shape[0]
    e = edge_index.shape[1]

    shift = max(n - 1, e // n).bit_length()
    packed = lax.sort(
        (edge_index[1] << shift) | edge_index[0], dimension=0)
    dsts = packed >> shift
    srcs = packed & ((1 << shift) - 1)
    offs = jnp.searchsorted(dsts, jnp.arange(n + 1, dtype=jnp.int32),
                            method="scan_unrolled").astype(jnp.int32)
    srcs_p = jnp.concatenate([srcs, jnp.zeros((64,), jnp.int32)])

    npt = _pad16((n + _NW - 1) // _NW)
    gmax = e // 16 + n
    deg = offs[1:] - offs[:-1]
    nblk = jnp.maximum((deg + 15) // 16, 1)
    boff = jnp.concatenate(
        [jnp.zeros((1,), jnp.int32),
         jnp.cumsum(nblk).astype(jnp.int32)])
    g = jnp.arange(gmax, dtype=jnp.int32)
    node = jnp.repeat(jnp.arange(n, dtype=jnp.int32), nblk,
                      total_repeat_length=gmax)
    st = g - boff[node]
    valid = g < boff[n]
    e0 = jnp.where(valid, offs[node] + st * 16, 0)
    rem = jnp.where(valid, jnp.clip(deg[node] - st * 16, 0, 16), 0)
    fl = jnp.where(valid,
                   (st == 0).astype(jnp.int32)
                   + 2 * (st == nblk[node] - 1).astype(jnp.int32), 0)
    meta = jnp.stack([node, e0, rem, fl], axis=1).reshape(-1)
    meta = jnp.concatenate([meta, jnp.zeros((4 * (_MW + 16),), jnp.int32)])
    tb = boff[jnp.minimum(
        jnp.arange(33, dtype=jnp.int32) * npt, n)].astype(jnp.int32)
    tb = jnp.concatenate([tb, jnp.full((15,), boff[n], jnp.int32)])

    def attn(hq, hkv, p, c_actual):
        cp = hq.shape[1]
        kfn, _ = _make_sc_attn(n, cp, c_actual)
        return kfn(hq, hkv, srcs_p, meta, tb)

    p = params

    def bn_ins(agg, sk, bn, c, relu_first=False):
        st = _stats_call(agg, sk, c, n, relu_first)
        return [(agg, "rows"), (sk, "rows"), (bn["g"], "full"),
                (bn["b"], "full"), (st, "full")]

    q1, kv1, s1 = _dense_call(lambda a: a, [(x, "rows")], p["conv1"],
                              _pad64(220), n)
    agg1 = attn(q1, kv1, p["conv1"], 220)

    q2, kv2, s2 = _dense_call(
        lambda a, s, g, b, st: jax.nn.relu(
            _bn_apply(a[:, :220] + s, g, b, st, n)),
        bn_ins(agg1, s1, p["bn1"], 220), p["conv2"], _pad64(150), n)
    agg2 = attn(q2, kv2, p["conv2"], 150)

    qs, kvs, ss = _dense_call(lambda a: a, [(x, "rows")], p["skip1"],
                              _pad64(150), n)
    aggs = attn(qs, kvs, p["skip1"], 150)

    q3, kv3, s3 = _dense_call(
        lambda a, s, g, b, st, ai, si: jax.nn.relu(
            _bn_apply(a[:, :150] + s, g, b, st, n) + ai[:, :150] + si),
        bn_ins(agg2, s2, p["bn2"], 150) + [(aggs, "rows"), (ss, "rows")],
        p["conv3"], _pad64(100), n)
    agg3 = attn(q3, kv3, p["conv3"], 100)

    q4, kv4, s4 = _dense_call(
        lambda a, s, g, b, st: jax.nn.relu(
            _bn_apply(a[:, :100] + s, g, b, st, n)),
        bn_ins(agg3, s3, p["bn3"], 100), p["conv4"], _pad64(60), n)
    agg4 = attn(q4, kv4, p["conv4"], 60)

    q5, kv5, s5 = _dense_call(
        lambda a, s, g, b, st: jax.nn.relu(
            _bn_apply(a[:, :60] + s, g, b, st, n)),
        bn_ins(agg4, s4, p["bn4"], 60), p["conv5"], _pad64(30), n)
    agg5 = attn(q5, kv5, p["conv5"], 30)

    q6, kv6, s6 = _dense_call(
        lambda a, s, g, b, st: _bn_apply(
            jax.nn.relu(a[:, :30] + s), g, b, st, n),
        bn_ins(agg5, s5, p["bn5"], 30, relu_first=True), p["conv6"],
        _pad64(10), n)
    agg6 = attn(q6, kv6, p["conv6"], 10)

    return _logsoftmax_call(agg6, s6, 10, n)


# speedup vs baseline: 1.5231x; 1.0670x over previous
import functools
import math

import jax
import jax.numpy as jnp
from jax import lax
from jax.experimental import pallas as pl
from jax.experimental.pallas import tpu as pltpu
from jax.experimental.pallas import tpu_sc as plsc

_NC = 2
_NS = 16
_NW = _NC * _NS
_L = 16


def _pad16(c):
    return ((c + 15) // 16) * 16


def _pad64(c):
    return ((c + 63) // 64) * 64


_MW = 120
_MB = 4 * (_MW + 12)


@functools.cache
def _make_sc_attn(n, cp, c_actual):
    tc = cp // 16
    npt = _pad16((n + _NW - 1) // _NW)
    inv = 1.0 / math.sqrt(float(c_actual))
    mesh = plsc.VectorSubcoreMesh(core_axis_name="c", subcore_axis_name="s")

    def body(q_hbm, kv_hbm, srcs_hbm, meta_hbm, tb_hbm, out_hbm,
             tbv, metabuf, srcwin, kvring, qring, aggbuf,
             swin_sem, kv_sem, q_sem):
        cid = lax.axis_index("c")
        sid = lax.axis_index("s")
        wid = sid * _NC + cid
        n_base = wid * npt
        pltpu.sync_copy(tb_hbm, tbv)
        bb = tbv[pl.ds(wid, 16)]
        b0, b1 = bb[0], bb[1]
        iota = lax.iota(jnp.int32, 16)
        nwin = lax.div(b1 - b0 + (_MW - 1), _MW)
        state0 = tuple(jnp.zeros((16,), jnp.float32) for _ in range(tc + 2))

        @pl.loop(0, nwin, init_carry=state0)
        def _win(w, state):
            gw0 = b0 + w * _MW
            cnt = jnp.minimum(_MW, b1 - gw0)
            fb_full = 4 * gw0
            fb = pl.multiple_of(fb_full & (-8), 8)
            off0 = fb_full - fb
            pltpu.sync_copy(meta_hbm.at[pl.ds(fb, _MB)], metabuf)

            def meta_at(idx):
                mv = metabuf[pl.ds(off0 + 4 * idx, 16)]
                return mv[0], mv[1], mv[2], mv[3]

            def issue_srcwin(l):
                _, e0l, _, _ = meta_at(l)
                basel = pl.multiple_of(e0l & (-8), 8)
                pltpu.make_async_copy(
                    srcs_hbm.at[pl.ds(basel, 32)],
                    srcwin.at[lax.bitwise_and(l, 7)], swin_sem).start()

            def wait_srcwin():
                pltpu.make_async_copy(
                    srcs_hbm.at[pl.ds(0, 32)], srcwin.at[0],
                    swin_sem).wait()

            def issue_kv_q(l):
                nodel, e0l, _, _ = meta_at(l)
                basel = pl.multiple_of(e0l & (-8), 8)
                slot = lax.bitwise_and(l, 3)
                sidx = srcwin[lax.bitwise_and(l, 7), pl.ds(e0l - basel, 16)]
                pltpu.make_async_copy(
                    kv_hbm.at[sidx], kvring.at[slot], kv_sem).start()
                pltpu.make_async_copy(
                    q_hbm.at[nodel], qring.at[slot], q_sem).start()

            for l in range(6):
                @pl.when(l < cnt)
                def _():
                    issue_srcwin(l)
            for l in range(3):
                @pl.when(l < cnt)
                def _():
                    wait_srcwin()
                    issue_kv_q(l)

            @pl.loop(0, cnt, init_carry=state)
            def _blk(i, carry):
                m, s = carry[0], carry[1]
                u = list(carry[2:])

                @pl.when(i + 6 < cnt)
                def _():
                    issue_srcwin(i + 6)

                @pl.when(i + 3 < cnt)
                def _():
                    wait_srcwin()
                    issue_kv_q(i + 3)

                pltpu.make_async_copy(
                    kv_hbm.at[pl.ds(0, 16)], kvring.at[0], kv_sem).wait()
                pltpu.make_async_copy(
                    q_hbm.at[0], qring.at[0], q_sem).wait()

                node, e0b, rem, fl = meta_at(i)
                slot = lax.bitwise_and(i, 3)
                first = lax.bitwise_and(fl, 1)
                last = fl // 2

                neg = jnp.full((16,), -3e38, jnp.float32)
                zero = jnp.zeros((16,), jnp.float32)
                m = jnp.where(first == 1, neg, m)
                s = jnp.where(first == 1, zero, s)
                u = [jnp.where(first == 1, zero, uu) for uu in u]

                qv = [qring[slot, pl.ds(t * 16, 16)] for t in range(tc)]
                mask = iota < rem
                alpha = jnp.full((16,), -1e30, jnp.float32)
                for r in range(16):
                    acc = qv[0] * kvring[slot, r, pl.ds(0, 16)]
                    for t in range(1, tc):
                        acc = acc + qv[t] * kvring[slot, r,
                                                   pl.ds(t * 16, 16)]
                    alpha = jnp.where(iota == r, jnp.sum(acc) * inv, alpha)
                alpha_m = jnp.where(mask, alpha, -1e30)
                m_new = jnp.maximum(m, jnp.max(alpha_m))
                scale = jnp.exp(m - m_new)
                evec = jnp.where(mask, jnp.exp(alpha - m_new), 0.0)
                s = s * scale + jnp.sum(evec)
                u = [uu * scale for uu in u]
                for r in range(16):
                    w_r = evec[r]
                    for t in range(tc):
                        u[t] = u[t] + w_r * kvring[slot, r,
                                                   pl.ds(cp + t * 16, 16)]

                @pl.when(last == 1)
                def _():
                    ln = node - n_base
                    row = lax.bitwise_and(ln, 15)
                    s_fin = s + 1e-16
                    for t in range(tc):
                        aggbuf[row, pl.ds(t * 16, 16)] = u[t] / s_fin

                    @pl.when(row == 15)
                    def _():
                        n0 = pl.multiple_of(n_base + ln - 15, 8)
                        pltpu.sync_copy(aggbuf, out_hbm.at[pl.ds(n0, 16)])

                return (m_new, s) + tuple(u)

            return _blk

    kfn = pl.kernel(
        body,
        out_type=jax.ShapeDtypeStruct((n, cp), jnp.float32),
        mesh=mesh,
        scratch_types=[
            pltpu.VMEM((48,), jnp.int32),
            pltpu.VMEM((_MB,), jnp.int32),
            pltpu.VMEM((8, 32), jnp.int32),
            pltpu.VMEM((4, 16, 2 * cp), jnp.float32),
            pltpu.VMEM((4, cp), jnp.float32),
            pltpu.VMEM((16, cp), jnp.float32),
            pltpu.SemaphoreType.DMA,
            pltpu.SemaphoreType.DMA,
            pltpu.SemaphoreType.DMA,
        ],
        compiler_params=pltpu.CompilerParams(needs_layout_passes=False),
    )
    return kfn, npt


_GRID = 10


def _stats_call(a, s, c, n, relu_first):
    blk = n // _GRID

    def body(a_ref, s_ref, o_ref):
        i = pl.program_id(0)
        z = a_ref[:, :c] + s_ref[:]
        if relu_first:
            z = jax.nn.relu(z)
        upd = jnp.concatenate(
            [jnp.sum(z, axis=0)[None], jnp.sum(z * z, axis=0)[None],
             jnp.zeros((6, c), jnp.float32)], axis=0)
        prev = jnp.where(i == 0, jnp.zeros((8, c), jnp.float32), o_ref[...])
        o_ref[...] = prev + upd

    return pl.pallas_call(
        body,
        grid=(_GRID,),
        in_specs=[
            pl.BlockSpec((blk, a.shape[1]), lambda i: (i, 0)),
            pl.BlockSpec((blk, c), lambda i: (i, 0)),
        ],
        out_specs=pl.BlockSpec((8, c), lambda i: (0, 0)),
        out_shape=jax.ShapeDtypeStruct((8, c), jnp.float32),
    )(a, s)


def _dense_call(prologue, ins, p, cp, n):
    fout = p["q"]["W"].shape[1]
    nw = len(ins)
    blk = n // _GRID

    def body(*refs):
        inr = refs[:nw]
        wq, bq, wk, bk, wv, bv, ws, bs = refs[nw:nw + 8]
        qo, kvo, so = refs[nw + 8:]
        h = prologue(*[r[...] for r in inr])
        q = jnp.dot(h, wq[:], preferred_element_type=jnp.float32) + bq[:]
        k = jnp.dot(h, wk[:], preferred_element_type=jnp.float32) + bk[:]
        v = jnp.dot(h, wv[:], preferred_element_type=jnp.float32) + bv[:]
        sk = jnp.dot(h, ws[:], preferred_element_type=jnp.float32) + bs[:]
        if cp > fout:
            z = jnp.zeros((blk, cp - fout), jnp.float32)
            qo[...] = jnp.concatenate([q, z], axis=1)
            kvo[...] = jnp.concatenate([k, z, v, z], axis=1)
        else:
            qo[...] = q
            kvo[...] = jnp.concatenate([k, v], axis=1)
        so[...] = sk

    def full_spec(arr):
        shp = arr.shape
        if len(shp) == 1:
            return pl.BlockSpec(shp, lambda i: (0,))
        return pl.BlockSpec(shp, lambda i: (0,) * len(shp))

    in_specs = []
    args = []
    for arr, kind in ins:
        args.append(arr)
        if kind == "rows":
            in_specs.append(
                pl.BlockSpec((blk, arr.shape[1]), lambda i: (i, 0)))
        else:
            in_specs.append(full_spec(arr))
    for w in [p["q"]["W"], p["q"]["b"], p["k"]["W"], p["k"]["b"],
              p["v"]["W"], p["v"]["b"], p["skip"]["W"], p["skip"]["b"]]:
        args.append(w)
        in_specs.append(full_spec(w))

    out_shape = [
        jax.ShapeDtypeStruct((n, cp), jnp.float32),
        jax.ShapeDtypeStruct((n, 2 * cp), jnp.float32),
        jax.ShapeDtypeStruct((n, fout), jnp.float32),
    ]
    out_specs = [
        pl.BlockSpec((blk, cp), lambda i: (i, 0)),
        pl.BlockSpec((blk, 2 * cp), lambda i: (i, 0)),
        pl.BlockSpec((blk, fout), lambda i: (i, 0)),
    ]
    return pl.pallas_call(
        body, grid=(_GRID,), in_specs=in_specs, out_specs=out_specs,
        out_shape=out_shape)(*args)


def _bn_apply(z, g, b, stats, n, eps=1e-5):
    mu = stats[0] / n
    var = stats[1] / n - mu * mu
    return (z - mu) / jnp.sqrt(var + eps) * g + b


def _logsoftmax_call(agg, sk, c, n):
    def body(a_ref, s_ref, o_ref):
        z = a_ref[:, :c] + s_ref[:]
        m = jnp.max(z, axis=1, keepdims=True)
        lse = m + jnp.log(jnp.sum(jnp.exp(z - m), axis=1, keepdims=True))
        o_ref[:] = z - lse

    return pl.pallas_call(
        body, out_shape=jax.ShapeDtypeStruct((n, c), jnp.float32))(agg, sk)


def kernel(x, edge_index, params):
    n = x.shape[0]
    e = edge_index.shape[1]

    shift = max(n - 1, e // n).bit_length()
    packed = lax.sort(
        (edge_index[1] << shift) | edge_index[0], dimension=0)
    dsts = packed >> shift
    srcs = packed & ((1 << shift) - 1)
    offs = jnp.searchsorted(dsts, jnp.arange(n + 1, dtype=jnp.int32),
                            method="scan_unrolled").astype(jnp.int32)
    srcs_p = jnp.concatenate([srcs, jnp.zeros((64,), jnp.int32)])

    npt = _pad16((n + _NW - 1) // _NW)
    gmax = e // 16 + n
    deg = offs[1:] - offs[:-1]
    nblk = jnp.maximum((deg + 15) // 16, 1)
    boff = jnp.concatenate(
        [jnp.zeros((1,), jnp.int32),
         jnp.cumsum(nblk).astype(jnp.int32)])
    g = jnp.arange(gmax, dtype=jnp.int32)
    node = jnp.repeat(jnp.arange(n, dtype=jnp.int32), nblk,
                      total_repeat_length=gmax)
    st = g - boff[node]
    valid = g < boff[n]
    e0 = jnp.where(valid, offs[node] + st * 16, 0)
    rem = jnp.where(valid, jnp.clip(deg[node] - st * 16, 0, 16), 0)
    fl = jnp.where(valid,
                   (st == 0).astype(jnp.int32)
                   + 2 * (st == nblk[node] - 1).astype(jnp.int32), 0)
    meta = jnp.stack([node, e0, rem, fl], axis=1).reshape(-1)
    meta = jnp.concatenate([meta, jnp.zeros((4 * (_MW + 16),), jnp.int32)])
    tb = boff[jnp.minimum(
        jnp.arange(33, dtype=jnp.int32) * npt, n)].astype(jnp.int32)
    tb = jnp.concatenate([tb, jnp.full((15,), boff[n], jnp.int32)])

    def attn(hq, hkv, p, c_actual):
        cp = hq.shape[1]
        kfn, _ = _make_sc_attn(n, cp, c_actual)
        return kfn(hq, hkv, srcs_p, meta, tb)

    p = params

    def bn_ins(agg, sk, bn, c, relu_first=False):
        st = _stats_call(agg, sk, c, n, relu_first)
        return [(agg, "rows"), (sk, "rows"), (bn["g"], "full"),
                (bn["b"], "full"), (st, "full")]

    q1, kv1, s1 = _dense_call(lambda a: a, [(x, "rows")], p["conv1"],
                              _pad64(220), n)
    agg1 = attn(q1, kv1, p["conv1"], 220)

    q2, kv2, s2 = _dense_call(
        lambda a, s, g, b, st: jax.nn.relu(
            _bn_apply(a[:, :220] + s, g, b, st, n)),
        bn_ins(agg1, s1, p["bn1"], 220), p["conv2"], _pad64(150), n)
    agg2 = attn(q2, kv2, p["conv2"], 150)

    qs, kvs, ss = _dense_call(lambda a: a, [(x, "rows")], p["skip1"],
                              _pad64(150), n)
    aggs = attn(qs, kvs, p["skip1"], 150)

    q3, kv3, s3 = _dense_call(
        lambda a, s, g, b, st, ai, si: jax.nn.relu(
            _bn_apply(a[:, :150] + s, g, b, st, n) + ai[:, :150] + si),
        bn_ins(agg2, s2, p["bn2"], 150) + [(aggs, "rows"), (ss, "rows")],
        p["conv3"], _pad64(100), n)
    agg3 = attn(q3, kv3, p["conv3"], 100)

    q4, kv4, s4 = _dense_call(
        lambda a, s, g, b, st: jax.nn.relu(
            _bn_apply(a[:, :100] + s, g, b, st, n)),
        bn_ins(agg3, s3, p["bn3"], 100), p["conv4"], _pad64(60), n)
    agg4 = attn(q4, kv4, p["conv4"], 60)

    q5, kv5, s5 = _dense_call(
        lambda a, s, g, b, st: jax.nn.relu(
            _bn_apply(a[:, :60] + s, g, b, st, n)),
        bn_ins(agg4, s4, p["bn4"], 60), p["conv5"], _pad64(30), n)
    agg5 = attn(q5, kv5, p["conv5"], 30)

    q6, kv6, s6 = _dense_call(
        lambda a, s, g, b, st: _bn_apply(
            jax.nn.relu(a[:, :30] + s), g, b, st, n),
        bn_ins(agg5, s5, p["bn5"], 30, relu_first=True), p["conv6"],
        _pad64(10), n)
    agg6 = attn(q6, kv6, p["conv6"], 10)

    return _logsoftmax_call(agg6, s6, 10, n)


# speedup vs baseline: 1.5626x; 1.0259x over previous
import functools
import math

import jax
import jax.numpy as jnp
from jax import lax
from jax.experimental import pallas as pl
from jax.experimental.pallas import tpu as pltpu
from jax.experimental.pallas import tpu_sc as plsc

_NC = 2
_NS = 16
_NW = _NC * _NS
_L = 16


def _pad16(c):
    return ((c + 15) // 16) * 16


def _pad64(c):
    return ((c + 63) // 64) * 64


_MW = 120
_MB = 4 * (_MW + 16)


@functools.cache
def _make_sc_attn(n, cp, c_actual):
    tc = cp // 16
    npt = _pad16((n + _NW - 1) // _NW)
    inv = 1.0 / math.sqrt(float(c_actual))
    mesh = plsc.VectorSubcoreMesh(core_axis_name="c", subcore_axis_name="s")

    def body(q_hbm, kv_hbm, srcs_hbm, meta_hbm, tb_hbm, out_hbm,
             tbv, metabuf, srcwin, kvring, qring, aggbuf,
             swin_sem, kv_sem, q_sem):
        cid = lax.axis_index("c")
        sid = lax.axis_index("s")
        wid = sid * _NC + cid
        n_base = wid * npt
        pltpu.sync_copy(tb_hbm, tbv)
        bb = tbv[pl.ds(wid, 16)]
        b0, b1 = bb[0], bb[1]
        iota = lax.iota(jnp.int32, 16)
        nwin = lax.div(b1 - b0 + (_MW - 1), _MW)
        state0 = tuple(jnp.zeros((16,), jnp.float32) for _ in range(tc + 2))

        @pl.loop(0, nwin, init_carry=state0)
        def _win(w, state):
            gw0 = b0 + w * _MW
            cnt = jnp.minimum(_MW, b1 - gw0)
            fb_full = 4 * gw0
            fb = pl.multiple_of(fb_full & (-8), 8)
            off0 = fb_full - fb
            pltpu.sync_copy(meta_hbm.at[pl.ds(fb, _MB)], metabuf)

            def meta_at(idx):
                mv = metabuf[pl.ds(off0 + 4 * idx, 16)]
                return mv[0], mv[1], mv[2], mv[3]

            def issue_srcwin(l):
                _, e0l, _, _ = meta_at(l)
                basel = pl.multiple_of(e0l & (-8), 8)
                pltpu.make_async_copy(
                    srcs_hbm.at[pl.ds(basel, 32)],
                    srcwin.at[lax.bitwise_and(l, 7)], swin_sem).start()

            def wait_srcwin():
                pltpu.make_async_copy(
                    srcs_hbm.at[pl.ds(0, 32)], srcwin.at[0],
                    swin_sem).wait()

            def issue_kv_q(l):
                nodel, e0l, _, _ = meta_at(l)
                basel = pl.multiple_of(e0l & (-8), 8)
                slot = lax.bitwise_and(l, 7)
                sidx = srcwin[lax.bitwise_and(l, 7), pl.ds(e0l - basel, 16)]
                pltpu.make_async_copy(
                    kv_hbm.at[sidx], kvring.at[slot], kv_sem).start()
                pltpu.make_async_copy(
                    q_hbm.at[nodel], qring.at[slot], q_sem).start()

            for l in range(8):
                @pl.when(l < cnt)
                def _():
                    issue_srcwin(l)
            for l in range(4):
                @pl.when(l < cnt)
                def _():
                    wait_srcwin()
                    issue_kv_q(l)

            @pl.loop(0, cnt, init_carry=state)
            def _blk(i, carry):
                m, s = carry[0], carry[1]
                u = list(carry[2:])

                @pl.when(i + 8 < cnt)
                def _():
                    issue_srcwin(i + 8)

                @pl.when(i + 4 < cnt)
                def _():
                    wait_srcwin()
                    issue_kv_q(i + 4)

                pltpu.make_async_copy(
                    kv_hbm.at[pl.ds(0, 16)], kvring.at[0], kv_sem).wait()
                pltpu.make_async_copy(
                    q_hbm.at[0], qring.at[0], q_sem).wait()

                node, e0b, rem, fl = meta_at(i)
                slot = lax.bitwise_and(i, 7)
                first = lax.bitwise_and(fl, 1)
                last = fl // 2

                neg = jnp.full((16,), -3e38, jnp.float32)
                zero = jnp.zeros((16,), jnp.float32)
                m = jnp.where(first == 1, neg, m)
                s = jnp.where(first == 1, zero, s)
                u = [jnp.where(first == 1, zero, uu) for uu in u]

                qv = [qring[slot, pl.ds(t * 16, 16)] for t in range(tc)]
                mask = iota < rem
                alpha = jnp.full((16,), -1e30, jnp.float32)
                for r in range(16):
                    acc = qv[0] * kvring[slot, r, pl.ds(0, 16)]
                    for t in range(1, tc):
                        acc = acc + qv[t] * kvring[slot, r,
                                                   pl.ds(t * 16, 16)]
                    alpha = jnp.where(iota == r, jnp.sum(acc) * inv, alpha)
                alpha_m = jnp.where(mask, alpha, -1e30)
                m_new = jnp.maximum(m, jnp.max(alpha_m))
                scale = jnp.exp(m - m_new)
                evec = jnp.where(mask, jnp.exp(alpha - m_new), 0.0)
                s = s * scale + jnp.sum(evec)
                u = [uu * scale for uu in u]
                for r in range(16):
                    w_r = evec[r]
                    for t in range(tc):
                        u[t] = u[t] + w_r * kvring[slot, r,
                                                   pl.ds(cp + t * 16, 16)]

                @pl.when(last == 1)
                def _():
                    ln = node - n_base
                    row = lax.bitwise_and(ln, 15)
                    s_fin = s + 1e-16
                    for t in range(tc):
                        aggbuf[row, pl.ds(t * 16, 16)] = u[t] / s_fin

                    @pl.when(row == 15)
                    def _():
                        n0 = pl.multiple_of(n_base + ln - 15, 8)
                        pltpu.sync_copy(aggbuf, out_hbm.at[pl.ds(n0, 16)])

                return (m_new, s) + tuple(u)

            return _blk

    kfn = pl.kernel(
        body,
        out_type=jax.ShapeDtypeStruct((n, cp), jnp.float32),
        mesh=mesh,
        scratch_types=[
            pltpu.VMEM((48,), jnp.int32),
            pltpu.VMEM((_MB,), jnp.int32),
            pltpu.VMEM((8, 32), jnp.int32),
            pltpu.VMEM((8, 16, 2 * cp), jnp.float32),
            pltpu.VMEM((8, cp), jnp.float32),
            pltpu.VMEM((16, cp), jnp.float32),
            pltpu.SemaphoreType.DMA,
            pltpu.SemaphoreType.DMA,
            pltpu.SemaphoreType.DMA,
        ],
        compiler_params=pltpu.CompilerParams(needs_layout_passes=False),
    )
    return kfn, npt


_GRID = 10


def _stats_call(a, s, c, n, relu_first):
    blk = n // _GRID

    def body(a_ref, s_ref, o_ref):
        i = pl.program_id(0)
        z = a_ref[:, :c] + s_ref[:]
        if relu_first:
            z = jax.nn.relu(z)
        upd = jnp.concatenate(
            [jnp.sum(z, axis=0)[None], jnp.sum(z * z, axis=0)[None],
             jnp.zeros((6, c), jnp.float32)], axis=0)
        prev = jnp.where(i == 0, jnp.zeros((8, c), jnp.float32), o_ref[...])
        o_ref[...] = prev + upd

    return pl.pallas_call(
        body,
        grid=(_GRID,),
        in_specs=[
            pl.BlockSpec((blk, a.shape[1]), lambda i: (i, 0)),
            pl.BlockSpec((blk, c), lambda i: (i, 0)),
        ],
        out_specs=pl.BlockSpec((8, c), lambda i: (0, 0)),
        out_shape=jax.ShapeDtypeStruct((8, c), jnp.float32),
    )(a, s)


def _dense_call(prologue, ins, p, cp, n):
    fout = p["q"]["W"].shape[1]
    nw = len(ins)
    blk = n // _GRID

    def body(*refs):
        inr = refs[:nw]
        wq, bq, wk, bk, wv, bv, ws, bs = refs[nw:nw + 8]
        qo, kvo, so = refs[nw + 8:]
        h = prologue(*[r[...] for r in inr])
        q = jnp.dot(h, wq[:], preferred_element_type=jnp.float32) + bq[:]
        k = jnp.dot(h, wk[:], preferred_element_type=jnp.float32) + bk[:]
        v = jnp.dot(h, wv[:], preferred_element_type=jnp.float32) + bv[:]
        sk = jnp.dot(h, ws[:], preferred_element_type=jnp.float32) + bs[:]
        if cp > fout:
            z = jnp.zeros((blk, cp - fout), jnp.float32)
            qo[...] = jnp.concatenate([q, z], axis=1)
            kvo[...] = jnp.concatenate([k, z, v, z], axis=1)
        else:
            qo[...] = q
            kvo[...] = jnp.concatenate([k, v], axis=1)
        so[...] = sk

    def full_spec(arr):
        shp = arr.shape
        if len(shp) == 1:
            return pl.BlockSpec(shp, lambda i: (0,))
        return pl.BlockSpec(shp, lambda i: (0,) * len(shp))

    in_specs = []
    args = []
    for arr, kind in ins:
        args.append(arr)
        if kind == "rows":
            in_specs.append(
                pl.BlockSpec((blk, arr.shape[1]), lambda i: (i, 0)))
        else:
            in_specs.append(full_spec(arr))
    for w in [p["q"]["W"], p["q"]["b"], p["k"]["W"], p["k"]["b"],
              p["v"]["W"], p["v"]["b"], p["skip"]["W"], p["skip"]["b"]]:
        args.append(w)
        in_specs.append(full_spec(w))

    out_shape = [
        jax.ShapeDtypeStruct((n, cp), jnp.float32),
        jax.ShapeDtypeStruct((n, 2 * cp), jnp.float32),
        jax.ShapeDtypeStruct((n, fout), jnp.float32),
    ]
    out_specs = [
        pl.BlockSpec((blk, cp), lambda i: (i, 0)),
        pl.BlockSpec((blk, 2 * cp), lambda i: (i, 0)),
        pl.BlockSpec((blk, fout), lambda i: (i, 0)),
    ]
    return pl.pallas_call(
        body, grid=(_GRID,), in_specs=in_specs, out_specs=out_specs,
        out_shape=out_shape)(*args)


def _bn_apply(z, g, b, stats, n, eps=1e-5):
    mu = stats[0] / n
    var = stats[1] / n - mu * mu
    return (z - mu) / jnp.sqrt(var + eps) * g + b


def _logsoftmax_call(agg, sk, c, n):
    def body(a_ref, s_ref, o_ref):
        z = a_ref[:, :c] + s_ref[:]
        m = jnp.max(z, axis=1, keepdims=True)
        lse = m + jnp.log(jnp.sum(jnp.exp(z - m), axis=1, keepdims=True))
        o_ref[:] = z - lse

    return pl.pallas_call(
        body, out_shape=jax.ShapeDtypeStruct((n, c), jnp.float32))(agg, sk)


def kernel(x, edge_index, params):
    n = x.shape[0]
    e = edge_index.shape[1]

    shift = max(n - 1, e // n).bit_length()
    packed = lax.sort(
        (edge_index[1] << shift) | edge_index[0], dimension=0)
    dsts = packed >> shift
    srcs = packed & ((1 << shift) - 1)
    offs = jnp.searchsorted(dsts, jnp.arange(n + 1, dtype=jnp.int32),
                            method="scan_unrolled").astype(jnp.int32)
    srcs_p = jnp.concatenate([srcs, jnp.zeros((64,), jnp.int32)])

    npt = _pad16((n + _NW - 1) // _NW)
    gmax = e // 16 + n
    deg = offs[1:] - offs[:-1]
    nblk = jnp.maximum((deg + 15) // 16, 1)
    boff = jnp.concatenate(
        [jnp.zeros((1,), jnp.int32),
         jnp.cumsum(nblk).astype(jnp.int32)])
    g = jnp.arange(gmax, dtype=jnp.int32)
    node = jnp.repeat(jnp.arange(n, dtype=jnp.int32), nblk,
                      total_repeat_length=gmax)
    st = g - boff[node]
    valid = g < boff[n]
    e0 = jnp.where(valid, offs[node] + st * 16, 0)
    rem = jnp.where(valid, jnp.clip(deg[node] - st * 16, 0, 16), 0)
    fl = jnp.where(valid,
                   (st == 0).astype(jnp.int32)
                   + 2 * (st == nblk[node] - 1).astype(jnp.int32), 0)
    meta = jnp.stack([node, e0, rem, fl], axis=1).reshape(-1)
    meta = jnp.concatenate([meta, jnp.zeros((4 * (_MW + 16),), jnp.int32)])
    tb = boff[jnp.minimum(
        jnp.arange(33, dtype=jnp.int32) * npt, n)].astype(jnp.int32)
    tb = jnp.concatenate([tb, jnp.full((15,), boff[n], jnp.int32)])

    def attn(hq, hkv, p, c_actual):
        cp = hq.shape[1]
        kfn, _ = _make_sc_attn(n, cp, c_actual)
        return kfn(hq, hkv, srcs_p, meta, tb)

    p = params

    def bn_ins(agg, sk, bn, c, relu_first=False):
        st = _stats_call(agg, sk, c, n, relu_first)
        return [(agg, "rows"), (sk, "rows"), (bn["g"], "full"),
                (bn["b"], "full"), (st, "full")]

    q1, kv1, s1 = _dense_call(lambda a: a, [(x, "rows")], p["conv1"],
                              _pad64(220), n)
    agg1 = attn(q1, kv1, p["conv1"], 220)

    q2, kv2, s2 = _dense_call(
        lambda a, s, g, b, st: jax.nn.relu(
            _bn_apply(a[:, :220] + s, g, b, st, n)),
        bn_ins(agg1, s1, p["bn1"], 220), p["conv2"], _pad64(150), n)
    agg2 = attn(q2, kv2, p["conv2"], 150)

    qs, kvs, ss = _dense_call(lambda a: a, [(x, "rows")], p["skip1"],
                              _pad64(150), n)
    aggs = attn(qs, kvs, p["skip1"], 150)

    q3, kv3, s3 = _dense_call(
        lambda a, s, g, b, st, ai, si: jax.nn.relu(
            _bn_apply(a[:, :150] + s, g, b, st, n) + ai[:, :150] + si),
        bn_ins(agg2, s2, p["bn2"], 150) + [(aggs, "rows"), (ss, "rows")],
        p["conv3"], _pad64(100), n)
    agg3 = attn(q3, kv3, p["conv3"], 100)

    q4, kv4, s4 = _dense_call(
        lambda a, s, g, b, st: jax.nn.relu(
            _bn_apply(a[:, :100] + s, g, b, st, n)),
        bn_ins(agg3, s3, p["bn3"], 100), p["conv4"], _pad64(60), n)
    agg4 = attn(q4, kv4, p["conv4"], 60)

    q5, kv5, s5 = _dense_call(
        lambda a, s, g, b, st: jax.nn.relu(
            _bn_apply(a[:, :60] + s, g, b, st, n)),
        bn_ins(agg4, s4, p["bn4"], 60), p["conv5"], _pad64(30), n)
    agg5 = attn(q5, kv5, p["conv5"], 30)

    q6, kv6, s6 = _dense_call(
        lambda a, s, g, b, st: _bn_apply(
            jax.nn.relu(a[:, :30] + s), g, b, st, n),
        bn_ins(agg5, s5, p["bn5"], 30, relu_first=True), p["conv6"],
        _pad64(10), n)
    agg6 = attn(q6, kv6, p["conv6"], 10)

    return _logsoftmax_call(agg6, s6, 10, n)
